# baseline (speedup 1.0000x reference)
import dataclasses
import functools
import math

import jax
import jax.numpy as jnp
from jax import lax
from jax.experimental import pallas as pl
from jax.experimental.pallas import tpu as pltpu
from jax.experimental.pallas import tpu_sc as plsc

_N_CLASSES = 32000
_PAD = 0
_FILL = 0.1 / (_N_CLASSES - 2)
_CONF = 0.9
_C_ROW = (_N_CLASSES - 2) * _FILL * math.log(_FILL) + _CONF * math.log(_CONF)

_N_ROWS = 4096
_BC = 640
_C0 = 30720
_NBJ = _C0 // _BC

_NW = 32
_RPW = _N_ROWS // _NW
_CC = 128
_NCH = (_N_CLASSES - _C0) // _CC
assert _NCH % 2 == 0


def _partial_body(x_ref, t_ref, o_ref, acc_ref, xt_ref, x0_ref):
    j = pl.program_id(0)

    @pl.when(j == 0)
    def _init():
        acc_ref[...] = jnp.zeros_like(acc_ref)
        xt_ref[...] = jnp.zeros_like(xt_ref)
        x0_ref[...] = x_ref[:, 0:1]

    t = t_ref[...]
    col0 = j * _BC
    blk = x_ref[...]
    rs = None
    xt = None
    for k in range(_BC // 128):
        sub = blk[:, k * 128:(k + 1) * 128]
        cols = col0 + k * 128 + jax.lax.broadcasted_iota(
            jnp.int32, (_N_ROWS, 128), 1)
        hit = jnp.where(cols == t, sub, 0.0)
        rs = sub if rs is None else rs + sub
        xt = hit if xt is None else xt + hit
    acc_ref[...] += rs
    xt_ref[...] += xt

    @pl.when(j == _NBJ - 1)
    def _fin():
        mask = (t != _PAD).astype(jnp.float32)
        cnt = jnp.sum(mask)
        sum_x0 = jnp.sum(mask * x0_ref[...])
        sum_rs = jnp.sum(mask * acc_ref[...])
        sum_xt = jnp.sum(mask * xt_ref[...])
        o_ref[0, 0] = (_C_ROW * cnt + _FILL * sum_x0 - _FILL * sum_rs
                       + (_FILL - _CONF) * sum_xt)


def _partial_call(x, t2):
    return pl.pallas_call(
        _partial_body,
        grid=(_NBJ,),
        in_specs=[
            pl.BlockSpec((_N_ROWS, _BC), lambda j: (0, j)),
            pl.BlockSpec((_N_ROWS, 1), lambda j: (0, 0)),
        ],
        out_specs=pl.BlockSpec(memory_space=pltpu.SMEM),
        out_shape=jax.ShapeDtypeStruct((1, 1), jnp.float32),
        scratch_shapes=[
            pltpu.VMEM((_N_ROWS, 128), jnp.float32),
            pltpu.VMEM((_N_ROWS, 128), jnp.float32),
            pltpu.VMEM((_N_ROWS, 1), jnp.float32),
        ],
    )(x, t2)


_SC_MESH = plsc.VectorSubcoreMesh(core_axis_name="c", subcore_axis_name="s")

_SC_PARAMS = pltpu.CompilerParams()
if "needs_layout_passes" in pltpu.CompilerParams.__dataclass_fields__:
    _SC_PARAMS = dataclasses.replace(_SC_PARAMS, needs_layout_passes=False)


@functools.partial(
    pl.kernel,
    out_type=jax.ShapeDtypeStruct((_NW, 16), jnp.float32),
    mesh=_SC_MESH,
    compiler_params=_SC_PARAMS,
    scratch_types=[
        pltpu.VMEM((_RPW,), jnp.int32),
        pltpu.VMEM((_RPW, _CC), jnp.float32),
        pltpu.VMEM((_RPW, _CC), jnp.float32),
        pltpu.VMEM((_RPW, 16), jnp.float32),
        pltpu.VMEM((_RPW,), jnp.float32),
        pltpu.VMEM((16,), jnp.float32),
        pltpu.SemaphoreType.DMA,
        pltpu.SemaphoreType.DMA((2,)),
    ],
)
def _sc_body(x_hbm, tgt_hbm, part_hbm, tgt_v, buf0, buf1, acc_v,
             xt_v, psum_v, sem_t, sem_b):
    wid = lax.axis_index("s") * 2 + lax.axis_index("c")
    base = pl.multiple_of(wid * _RPW, _RPW)
    pltpu.async_copy(tgt_hbm.at[pl.ds(base, _RPW)], tgt_v, sem_t).wait()

    @pl.loop(0, _RPW)
    def _zero(r):
        acc_v[r, pl.ds(0, 16)] = jnp.zeros((16,), jnp.float32)

    for g in range(_RPW // 16):
        xt_v[pl.ds(g * 16, 16)] = jnp.zeros((16,), jnp.float32)
    bufs = (buf0, buf1)

    def _issue(j, b):
        col = pl.multiple_of(_C0 + j * _CC, 128)
        pltpu.make_async_copy(
            x_hbm.at[pl.ds(base, _RPW), pl.ds(col, _CC)],
            bufs[b], sem_b.at[b],
        ).start()

    def _process(j, b):
        pltpu.make_async_copy(
            x_hbm.at[pl.ds(base, _RPW), pl.ds(0, _CC)],
            bufs[b], sem_b.at[b],
        ).wait()
        buf = bufs[b]
        chunk_lo = _C0 + j * _CC
        for g in range(_RPW // 16):
            t16 = tgt_v[pl.ds(g * 16, 16)]
            rows = g * 16 + lax.iota(jnp.int32, 16)
            ci = t16 - chunk_lo
            valid = (ci >= 0) & (ci < _CC)
            ci_c = jnp.minimum(jnp.maximum(ci, 0), _CC - 1)
            lg = plsc.load_gather(buf, [rows, ci_c])
            xt_v[pl.ds(g * 16, 16)] += jnp.where(valid, lg, 0.0)

        @pl.loop(0, _RPW)
        def _rows(r):
            part = None
            for v in range(_CC // 16):
                vec = buf[r, pl.ds(v * 16, 16)]
                part = vec if part is None else part + vec
            acc_v[r, pl.ds(0, 16)] += part

    _issue(0, 0)
    _issue(1, 1)

    @pl.loop(0, _NCH - 2, step=2)
    def _steady(j0):
        _process(j0, 0)
        _issue(j0 + 2, 0)
        _process(j0 + 1, 1)
        _issue(j0 + 3, 1)

    _process(_NCH - 2, 0)
    _process(_NCH - 1, 1)

    psum = jnp.zeros((16,), jnp.float32)
    for g in range(_RPW // 16):
        rows = g * 16 + lax.iota(jnp.int32, 16)
        tot = None
        for c in range(16):
            colv = plsc.load_gather(acc_v, [rows, jnp.full((16,), c,
                                                           jnp.int32)])
            tot = colv if tot is None else tot + colv
        t16 = tgt_v[pl.ds(g * 16, 16)]
        xt16 = xt_v[pl.ds(g * 16, 16)]
        contrib = -_FILL * tot + (_FILL - _CONF) * xt16
        psum = psum + jnp.where(t16 != _PAD, contrib, 0.0)
    psum_v[...] = psum

    pltpu.sync_copy(psum_v, part_hbm.at[wid])


def kernel(x, target):
    tgt = target.astype(jnp.int32)
    t2 = tgt.reshape(_N_ROWS, 1)
    a = _partial_call(x, t2)
    part = _sc_body(x, tgt)
    return a[0, 0] + jnp.sum(part)


# speedup vs baseline: 1.0051x; 1.0051x over previous
import dataclasses
import functools
import math

import jax
import jax.numpy as jnp
from jax import lax
from jax.experimental import pallas as pl
from jax.experimental.pallas import tpu as pltpu
from jax.experimental.pallas import tpu_sc as plsc

_N_CLASSES = 32000
_PAD = 0
_FILL = 0.1 / (_N_CLASSES - 2)
_CONF = 0.9
_C_ROW = (_N_CLASSES - 2) * _FILL * math.log(_FILL) + _CONF * math.log(_CONF)

_N_ROWS = 4096
_BC = 640
_C0 = 30720
_NBJ = _C0 // _BC

_NW = 32
_RPW = _N_ROWS // _NW
_CC = 128
_NCH = (_N_CLASSES - _C0) // _CC
assert _NCH % 2 == 0


def _partial_body(x_ref, t_ref, o_ref, acc_ref, xt_ref, x0_ref):
    j = pl.program_id(0)

    @pl.when(j == 0)
    def _init():
        acc_ref[...] = jnp.zeros_like(acc_ref)
        xt_ref[...] = jnp.zeros_like(xt_ref)
        x0_ref[...] = x_ref[:, 0:1]

    t = t_ref[...]
    col0 = j * _BC
    blk = x_ref[...]
    rs = None
    xt = None
    for k in range(_BC // 128):
        sub = blk[:, k * 128:(k + 1) * 128]
        cols = col0 + k * 128 + jax.lax.broadcasted_iota(
            jnp.int32, (_N_ROWS, 128), 1)
        hit = jnp.where(cols == t, sub, 0.0)
        rs = sub if rs is None else rs + sub
        xt = hit if xt is None else xt + hit
    acc_ref[...] += rs
    xt_ref[...] += xt

    @pl.when(j == _NBJ - 1)
    def _fin():
        mask = (t != _PAD).astype(jnp.float32)
        cnt = jnp.sum(mask)
        sum_x0 = jnp.sum(mask * x0_ref[...])
        sum_rs = jnp.sum(mask * acc_ref[...])
        sum_xt = jnp.sum(mask * xt_ref[...])
        o_ref[0, 0] = (_C_ROW * cnt + _FILL * sum_x0 - _FILL * sum_rs
                       + (_FILL - _CONF) * sum_xt)


def _partial_call(x, t2):
    return pl.pallas_call(
        _partial_body,
        grid=(_NBJ,),
        in_specs=[
            pl.BlockSpec((_N_ROWS, _BC), lambda j: (0, j)),
            pl.BlockSpec((_N_ROWS, 1), lambda j: (0, 0)),
        ],
        out_specs=pl.BlockSpec(memory_space=pltpu.SMEM),
        out_shape=jax.ShapeDtypeStruct((1, 1), jnp.float32),
        scratch_shapes=[
            pltpu.VMEM((_N_ROWS, 128), jnp.float32),
            pltpu.VMEM((_N_ROWS, 128), jnp.float32),
            pltpu.VMEM((_N_ROWS, 1), jnp.float32),
        ],
    )(x, t2)


_SC_MESH = plsc.VectorSubcoreMesh(core_axis_name="c", subcore_axis_name="s")

_SC_PARAMS = pltpu.CompilerParams()
if "needs_layout_passes" in pltpu.CompilerParams.__dataclass_fields__:
    _SC_PARAMS = dataclasses.replace(_SC_PARAMS, needs_layout_passes=False)


@functools.partial(
    pl.kernel,
    out_type=jax.ShapeDtypeStruct((_NW, 16), jnp.float32),
    mesh=_SC_MESH,
    compiler_params=_SC_PARAMS,
    scratch_types=[
        pltpu.VMEM((_RPW,), jnp.int32),
        pltpu.VMEM((_RPW, _CC), jnp.float32),
        pltpu.VMEM((_RPW, _CC), jnp.float32),
        pltpu.VMEM((_RPW, 16), jnp.float32),
        pltpu.VMEM((_RPW,), jnp.float32),
        pltpu.VMEM((16,), jnp.float32),
        pltpu.SemaphoreType.DMA,
        pltpu.SemaphoreType.DMA((2,)),
    ],
)
def _sc_body(x_hbm, tgt_hbm, part_hbm, tgt_v, buf0, buf1, acc_v,
             xt_v, psum_v, sem_t, sem_b):
    wid = lax.axis_index("s") * 2 + lax.axis_index("c")
    base = pl.multiple_of(wid * _RPW, _RPW)
    pltpu.async_copy(tgt_hbm.at[pl.ds(base, _RPW)], tgt_v, sem_t).wait()

    @pl.loop(0, _RPW)
    def _zero(r):
        acc_v[r, pl.ds(0, 16)] = jnp.zeros((16,), jnp.float32)

    for g in range(_RPW // 16):
        xt_v[pl.ds(g * 16, 16)] = jnp.zeros((16,), jnp.float32)
    bufs = (buf0, buf1)

    def _issue(j, b):
        col = pl.multiple_of(_C0 + j * _CC, 128)
        pltpu.make_async_copy(
            x_hbm.at[pl.ds(base, _RPW), pl.ds(col, _CC)],
            bufs[b], sem_b.at[b],
        ).start()

    def _process(j, b):
        pltpu.make_async_copy(
            x_hbm.at[pl.ds(base, _RPW), pl.ds(0, _CC)],
            bufs[b], sem_b.at[b],
        ).wait()
        buf = bufs[b]
        chunk_lo = _C0 + j * _CC
        for g in range(_RPW // 16):
            t16 = tgt_v[pl.ds(g * 16, 16)]
            rows = g * 16 + lax.iota(jnp.int32, 16)
            ci = t16 - chunk_lo
            valid = (ci >= 0) & (ci < _CC)
            ci_c = jnp.minimum(jnp.maximum(ci, 0), _CC - 1)
            lg = plsc.load_gather(buf, [rows, ci_c])
            xt_v[pl.ds(g * 16, 16)] += jnp.where(valid, lg, 0.0)

        @pl.loop(0, _RPW)
        def _rows(r):
            part = None
            for v in range(_CC // 16):
                vec = buf[r, pl.ds(v * 16, 16)]
                part = vec if part is None else part + vec
            acc_v[r, pl.ds(0, 16)] += part

    _issue(0, 0)
    _issue(1, 1)

    @pl.loop(0, _NCH - 2, step=2)
    def _steady(j0):
        _process(j0, 0)
        _issue(j0 + 2, 0)
        _process(j0 + 1, 1)
        _issue(j0 + 3, 1)

    _process(_NCH - 2, 0)
    _process(_NCH - 1, 1)

    psum = jnp.zeros((16,), jnp.float32)
    for g in range(_RPW // 16):
        rows = g * 16 + lax.iota(jnp.int32, 16)
        tot = None
        for c in range(16):
            colv = plsc.load_gather(acc_v, [rows, jnp.full((16,), c,
                                                           jnp.int32)])
            tot = colv if tot is None else tot + colv
        t16 = tgt_v[pl.ds(g * 16, 16)]
        xt16 = xt_v[pl.ds(g * 16, 16)]
        contrib = -_FILL * tot + (_FILL - _CONF) * xt16
        psum = psum + jnp.where(t16 != _PAD, contrib, 0.0)
    psum_v[...] = psum

    pltpu.sync_copy(psum_v, part_hbm.at[wid])


def _combine_body(a_ref, p_ref, o_ref):
    o_ref[0, 0] = a_ref[0, 0] + jnp.sum(p_ref[...])


def _combine_call(a, part):
    return pl.pallas_call(
        _combine_body,
        in_specs=[
            pl.BlockSpec(memory_space=pltpu.SMEM),
            pl.BlockSpec((_NW, 16), lambda: (0, 0)),
        ],
        out_specs=pl.BlockSpec(memory_space=pltpu.SMEM),
        out_shape=jax.ShapeDtypeStruct((1, 1), jnp.float32),
    )(a, part)


def kernel(x, target):
    tgt = target.astype(jnp.int32)
    t2 = tgt.reshape(_N_ROWS, 1)
    a = _partial_call(x, t2)
    part = _sc_body(x, tgt)
    return _combine_call(a, part)[0, 0]
